# baseline (speedup 1.0000x reference)
import functools
import jax
import jax.numpy as jnp
from jax import lax
from jax.experimental import pallas as pl
from jax.experimental.pallas import tpu as pltpu
from jax.experimental.pallas import tpu_sc as plsc

N = 10000
E = 320000
D = 128
NG = D // 16

NC = 2
NS = 16
NW = NC * NS
CH = 128
NCHUNK = E // CH
CPT = 80
NCHP = CPT * NW
NP = 10240
ROWS_PT = NP // NS


def _gate_chunk(rows_ref, ew_ref, c, wvecs):
    def body(k, _):
        win = ew_ref[c, pl.ds(k, 16)]
        s = jnp.full((16,), -win[0])
        for j in range(NG):
            e = jnp.exp(s * wvecs[j])
            g = 1.0 / (1.0 + e)
            sl = pl.ds(j * 16, 16)
            rows_ref[k, sl] = rows_ref[k, sl] * g
        return 0
    lax.fori_loop(0, CH, body, 0, unroll=2)


def _copy_row(src2d, c, dst1d):
    for j in range(NG):
        sl = pl.ds(j * 16, 16)
        dst1d[sl] = src2d[c, sl]


def _sc_body(h_hbm, col_hbm, row_hbm, ew_hbm, wedge_hbm, out_hbm,
             col2d, row2d, ew2d, rows0, rows1,
             col0, col1, rowi0, rowi1, wedge_v, agg_sh, gs0, gs1):
    sid = lax.axis_index("s")
    cid = lax.axis_index("c")
    wid = sid * NC + cid
    cstart = wid * CPT

    rows = [rows0, rows1]
    cols = [col0, col1]
    rowis = [rowi0, rowi1]
    gsems = [gs0, gs1]

    pltpu.sync_copy(col_hbm.at[pl.ds(cstart, CPT)], col2d)
    pltpu.sync_copy(row_hbm.at[pl.ds(cstart, CPT)], row2d)
    pltpu.sync_copy(ew_hbm.at[pl.ds(cstart, CPT)], ew2d.at[pl.ds(0, CPT)])
    pltpu.sync_copy(wedge_hbm, wedge_v)

    def zbody(i, _):
        z = jnp.zeros((16,), jnp.float32)
        for j in range(NG):
            rows0[i, pl.ds(j * 16, 16)] = z
        return 0
    lax.fori_loop(0, CH, zbody, 0)
    for z in range(ROWS_PT // CH):
        pltpu.sync_copy(rows0, agg_sh.at[pl.ds(sid * ROWS_PT + z * CH, CH)])
    plsc.subcore_barrier()

    wvecs = [wedge_v[pl.ds(j * 16, 16)] for j in range(NG)]

    def gather(c, b):
        _copy_row(col2d, c, cols[b])
        return pltpu.async_copy(h_hbm.at[cols[b]], rows[b], gsems[b])

    def chunk(c, _):
        gather(c, 0).wait()
        _gate_chunk(rows[0], ew2d, c, wvecs)
        _copy_row(row2d, c, rowis[0])
        pltpu.sync_copy(rows[0], agg_sh.at[rowis[0]], add=True)
        return 0
    lax.fori_loop(0, CPT, chunk, 0)

    plsc.subcore_barrier()
    pltpu.sync_copy(agg_sh.at[pl.ds(sid * ROWS_PT, ROWS_PT)],
                    out_hbm.at[cid, pl.ds(sid * ROWS_PT, ROWS_PT)])


@jax.jit
def _sc_aggregate(h_neigh, col, row, ew, w_edge):
    mesh = plsc.VectorSubcoreMesh(core_axis_name="c", subcore_axis_name="s")
    f = pl.kernel(
        _sc_body,
        out_type=jax.ShapeDtypeStruct((NC, NP, D), jnp.float32),
        mesh=mesh,
        scratch_types=[
            pltpu.VMEM((CPT, CH), jnp.int32),
            pltpu.VMEM((CPT, CH), jnp.int32),
            pltpu.VMEM((CPT + 1, CH), jnp.float32),
            pltpu.VMEM((CH, D), jnp.float32),
            pltpu.VMEM((CH, D), jnp.float32),
            pltpu.VMEM((CH,), jnp.int32),
            pltpu.VMEM((CH,), jnp.int32),
            pltpu.VMEM((CH,), jnp.int32),
            pltpu.VMEM((CH,), jnp.int32),
            pltpu.VMEM((D,), jnp.float32),
            pltpu.VMEM_SHARED((NP, D), jnp.float32),
            pltpu.SemaphoreType.DMA,
            pltpu.SemaphoreType.DMA,
        ],
    )
    return f(h_neigh, col, row, ew, w_edge)


def _mm_body(x_ref, wt_ref, b_ref, o_ref):
    o_ref[...] = (
        jnp.dot(x_ref[...], wt_ref[...], preferred_element_type=jnp.float32)
        + b_ref[...]
    )


@jax.jit
def _mm(x, wt, b):
    bm = 400
    return pl.pallas_call(
        _mm_body,
        grid=(N // bm,),
        in_specs=[
            pl.BlockSpec((bm, D), lambda i: (i, 0)),
            pl.BlockSpec((D, D), lambda i: (0, 0)),
            pl.BlockSpec((1, D), lambda i: (0, 0)),
        ],
        out_specs=pl.BlockSpec((bm, D), lambda i: (i, 0)),
        out_shape=jax.ShapeDtypeStruct((N, D), jnp.float32),
    )(x, wt, b)


def _final_body(x_ref, wt_ref, b_ref, a_ref, g_ref, be_ref, o_ref):
    h = (
        jnp.dot(x_ref[...], wt_ref[...], preferred_element_type=jnp.float32)
        + b_ref[...]
        + a_ref[0]
        + a_ref[1]
    )
    mean = jnp.mean(h, axis=-1, keepdims=True)
    cent = h - mean
    var = jnp.mean(cent * cent, axis=-1, keepdims=True)
    y = cent * lax.rsqrt(var + 1e-5) * g_ref[...] + be_ref[...]
    o_ref[...] = jnp.where(y >= 0, y, 0.2 * y)


@jax.jit
def _final(x, wt, b, agg, gamma, beta):
    bm = 400
    return pl.pallas_call(
        _final_body,
        grid=(N // bm,),
        in_specs=[
            pl.BlockSpec((bm, D), lambda i: (i, 0)),
            pl.BlockSpec((D, D), lambda i: (0, 0)),
            pl.BlockSpec((1, D), lambda i: (0, 0)),
            pl.BlockSpec((NC, bm, D), lambda i: (0, i, 0)),
            pl.BlockSpec((1, D), lambda i: (0, 0)),
            pl.BlockSpec((1, D), lambda i: (0, 0)),
        ],
        out_specs=pl.BlockSpec((bm, D), lambda i: (i, 0)),
        out_shape=jax.ShapeDtypeStruct((N, D), jnp.float32),
    )(x, wt, b, agg, gamma, beta)


def kernel(x, edge_index, edge_weight, W_self, b_self, W_neigh, b_neigh,
           w_edge, ln_gamma, ln_beta):
    npad = NCHP - NCHUNK
    row = jnp.concatenate(
        [edge_index[0].astype(jnp.int32).reshape(NCHUNK, CH),
         jnp.full((npad, CH), N, jnp.int32)])
    col = jnp.concatenate(
        [edge_index[1].astype(jnp.int32).reshape(NCHUNK, CH),
         jnp.zeros((npad, CH), jnp.int32)])
    ew = jnp.concatenate(
        [edge_weight.reshape(NCHUNK, CH), jnp.zeros((npad, CH), jnp.float32)])
    h_neigh = _mm(x, W_neigh.T, b_neigh.reshape(1, D))
    agg = _sc_aggregate(h_neigh, col, row, ew, w_edge)
    return _final(x, W_self.T, b_self.reshape(1, D), agg,
                  ln_gamma.reshape(1, D), ln_beta.reshape(1, D))
